# Initial kernel scaffold; baseline (speedup 1.0000x reference)
import functools

import jax
import jax.numpy as jnp
from jax import lax
from jax.experimental import pallas as pl
from jax.experimental.pallas import tpu as pltpu
from jax.experimental.pallas import tpu_sc as plsc

N_NODES = 10000
N_EDGES = 320000
D_FEAT = 128
HIDDEN = 128
N_CLASSES = 40
C_PAD = 48

NC = 2
NS = 16
NW = NC * NS
N_PAD = 10240
ROWS_PER_SUB = N_PAD // NS
EPW = N_EDGES // NW
EB = 80
NCHUNK = EPW // EB
DEG_W = 16

_MESH = plsc.VectorSubcoreMesh(core_axis_name="c", subcore_axis_name="s")


def _zero_acc(zbuf, acc, s, width):
    def zrow(e, carry):
        for j in range(width // 16):
            zbuf[e, pl.ds(j * 16, 16)] = jnp.zeros((16,), jnp.float32)
        return carry

    lax.fori_loop(0, EB, zrow, 0)

    def zcopy(i, carry):
        pltpu.sync_copy(zbuf, acc.at[pl.ds(s * ROWS_PER_SUB + i * EB, EB), :])
        return carry

    lax.fori_loop(0, ROWS_PER_SUB // EB, zcopy, 0)


@functools.partial(
    pl.kernel,
    out_type=jax.ShapeDtypeStruct((NC, N_PAD, DEG_W), jnp.float32),
    mesh=_MESH,
    scratch_types=[
        pltpu.VMEM((EB,), jnp.float32),
        pltpu.VMEM((EB,), jnp.int32),
        pltpu.VMEM((EB, DEG_W), jnp.float32),
        pltpu.VMEM_SHARED((N_PAD, DEG_W), jnp.float32),
    ],
)
def _deg_kernel(ew_hbm, dst_hbm, out_hbm, ewv, dstv, rows, acc):
    c = lax.axis_index("c")
    s = lax.axis_index("s")
    wid = s * NC + c
    _zero_acc(rows, acc, s, DEG_W)
    plsc.subcore_barrier()

    def chunk(i, carry):
        off = wid * EPW + i * EB
        pltpu.sync_copy(ew_hbm.at[pl.ds(off, EB)], ewv)
        pltpu.sync_copy(dst_hbm.at[pl.ds(off, EB)], dstv)

        def fill(e, c2):
            w = ewv[e]
            rows[e, :] = jnp.full((DEG_W,), w, jnp.float32)
            return c2

        lax.fori_loop(0, EB, fill, 0)
        pltpu.sync_copy(rows, acc.at[dstv], add=True)
        return carry

    lax.fori_loop(0, NCHUNK, chunk, 0)
    plsc.subcore_barrier()
    pltpu.sync_copy(
        acc.at[pl.ds(s * ROWS_PER_SUB, ROWS_PER_SUB), :],
        out_hbm.at[c, pl.ds(s * ROWS_PER_SUB, ROWS_PER_SUB), :],
    )


def _make_msg_kernel(width):
    @functools.partial(
        pl.kernel,
        out_type=jax.ShapeDtypeStruct((NC, N_PAD, width), jnp.float32),
        mesh=_MESH,
        scratch_types=[
            pltpu.VMEM((EB,), jnp.int32),
            pltpu.VMEM((EB,), jnp.int32),
            pltpu.VMEM((EB,), jnp.float32),
            pltpu.VMEM((EB, width), jnp.float32),
            pltpu.VMEM_SHARED((N_PAD, width), jnp.float32),
            pltpu.SemaphoreType.DMA,
        ],
    )
    def msg(g_hbm, src_hbm, dst_hbm, ew_hbm, out_hbm,
            srcv, dstv, ewv, rows, acc, sem):
        c = lax.axis_index("c")
        s = lax.axis_index("s")
        wid = s * NC + c
        _zero_acc(rows, acc, s, width)
        plsc.subcore_barrier()

        def chunk(i, carry):
            off = wid * EPW + i * EB
            pltpu.sync_copy(src_hbm.at[pl.ds(off, EB)], srcv)
            pltpu.sync_copy(dst_hbm.at[pl.ds(off, EB)], dstv)
            pltpu.sync_copy(ew_hbm.at[pl.ds(off, EB)], ewv)
            pltpu.async_copy(g_hbm.at[srcv], rows, sem).wait()

            def scale(e, c2):
                w = ewv[e]
                for j in range(width // 16):
                    rows[e, pl.ds(j * 16, 16)] = rows[e, pl.ds(j * 16, 16)] * w
                return c2

            lax.fori_loop(0, EB, scale, 0)
            pltpu.sync_copy(rows, acc.at[dstv], add=True)
            return carry

        lax.fori_loop(0, NCHUNK, chunk, 0)
        plsc.subcore_barrier()
        pltpu.sync_copy(
            acc.at[pl.ds(s * ROWS_PER_SUB, ROWS_PER_SUB), :],
            out_hbm.at[c, pl.ds(s * ROWS_PER_SUB, ROWS_PER_SUB), :],
        )

    return msg


_msg128 = _make_msg_kernel(HIDDEN)
_msg48 = _make_msg_kernel(C_PAD)


_BN = 2000


def _mm1_body(x_ref, w_ref, o_ref):
    o_ref[...] = jnp.dot(x_ref[...], w_ref[...],
                         preferred_element_type=jnp.float32)


def _scale1_body(d0_ref, d1_ref, h_ref, dis_ref, g_ref):
    dis = lax.rsqrt(d0_ref[...] + d1_ref[...] + 1.0)
    dis_ref[...] = dis
    g_ref[...] = dis * h_ref[...]


def _mid_body(a0_ref, a1_ref, g1_ref, dis_ref, b1_ref, w2_ref, g2_ref):
    dis = dis_ref[...]
    h = dis * (a0_ref[...] + a1_ref[...] + g1_ref[...]) + b1_ref[...]
    h = jnp.maximum(h, 0.0)
    h2 = jnp.dot(h, w2_ref[...], preferred_element_type=jnp.float32)
    g2_ref[...] = dis * h2


def _final_body(a0_ref, a1_ref, g2_ref, dis_ref, b2_ref, o_ref):
    o48 = dis_ref[...] * (a0_ref[...] + a1_ref[...] + g2_ref[...]) + b2_ref[...]
    o = o48[:, :N_CLASSES]
    m = jnp.max(o, axis=1, keepdims=True)
    lse = m + jnp.log(jnp.sum(jnp.exp(o - m), axis=1, keepdims=True))
    o_ref[...] = o - lse


def _rows_spec(width):
    return pl.BlockSpec((_BN, width), lambda i: (i, 0))


def _full_spec(shape):
    return pl.BlockSpec(shape, lambda i: tuple(0 for _ in shape))


def kernel(x, edge_index, edge_weight, W1, b1, W2, b2):
    src = edge_index[0].astype(jnp.int32)
    dst = edge_index[1].astype(jnp.int32)
    ew = edge_weight.astype(jnp.float32)

    deg_parts = _deg_kernel(ew, dst)
    d0 = deg_parts[0, :N_NODES, 0:1]
    d1 = deg_parts[1, :N_NODES, 0:1]

    grid = N_NODES // _BN
    h1 = pl.pallas_call(
        _mm1_body,
        grid=(grid,),
        in_specs=[_rows_spec(D_FEAT), _full_spec((D_FEAT, HIDDEN))],
        out_specs=_rows_spec(HIDDEN),
        out_shape=jax.ShapeDtypeStruct((N_NODES, HIDDEN), jnp.float32),
    )(x, W1)

    dis, g1 = pl.pallas_call(
        _scale1_body,
        grid=(grid,),
        in_specs=[_rows_spec(1), _rows_spec(1), _rows_spec(HIDDEN)],
        out_specs=[_rows_spec(1), _rows_spec(HIDDEN)],
        out_shape=[
            jax.ShapeDtypeStruct((N_NODES, 1), jnp.float32),
            jax.ShapeDtypeStruct((N_NODES, HIDDEN), jnp.float32),
        ],
    )(d0, d1, h1)

    a1 = _msg128(g1, src, dst, ew)

    W2p = jnp.pad(W2, ((0, 0), (0, C_PAD - N_CLASSES)))
    b1r = b1.reshape(1, HIDDEN)
    b2r = jnp.pad(b2, (0, C_PAD - N_CLASSES)).reshape(1, C_PAD)

    g2 = pl.pallas_call(
        _mid_body,
        grid=(grid,),
        in_specs=[
            _rows_spec(HIDDEN), _rows_spec(HIDDEN), _rows_spec(HIDDEN),
            _rows_spec(1), _full_spec((1, HIDDEN)),
            _full_spec((HIDDEN, C_PAD)),
        ],
        out_specs=_rows_spec(C_PAD),
        out_shape=jax.ShapeDtypeStruct((N_NODES, C_PAD), jnp.float32),
    )(a1[0, :N_NODES], a1[1, :N_NODES], g1, dis, b1r, W2p)

    a2 = _msg48(g2, src, dst, ew)

    out = pl.pallas_call(
        _final_body,
        grid=(grid,),
        in_specs=[
            _rows_spec(C_PAD), _rows_spec(C_PAD), _rows_spec(C_PAD),
            _rows_spec(1), _full_spec((1, C_PAD)),
        ],
        out_specs=_rows_spec(N_CLASSES),
        out_shape=jax.ShapeDtypeStruct((N_NODES, N_CLASSES), jnp.float32),
    )(a2[0, :N_NODES], a2[1, :N_NODES], g2, dis, b2r)

    return out


# speedup vs baseline: 10.5682x; 10.5682x over previous
import functools

import jax
import jax.numpy as jnp
from jax import lax
from jax.experimental import pallas as pl
from jax.experimental.pallas import tpu as pltpu
from jax.experimental.pallas import tpu_sc as plsc

N_NODES = 10000
N_EDGES = 320000
D_FEAT = 128
HIDDEN = 128
N_CLASSES = 40
C_PAD = 48

NC = 2
NS = 16
NW = NC * NS
N_PAD = 10240
ROWS_PER_SUB = N_PAD // NS
EPW = N_EDGES // NW
EB = 80
NCHUNK = EPW // EB
DEG_W = 16

_MESH = plsc.VectorSubcoreMesh(core_axis_name="c", subcore_axis_name="s")
_SC_PARAMS = pltpu.CompilerParams(use_tc_tiling_on_sc=False)


def _zero_acc(zbuf, acc, s, width):
    def zrow(e, carry):
        for j in range(width // 16):
            zbuf[e, pl.ds(j * 16, 16)] = jnp.zeros((16,), jnp.float32)
        return carry

    lax.fori_loop(0, EB, zrow, 0)

    def zcopy(i, carry):
        pltpu.sync_copy(zbuf, acc.at[pl.ds(s * ROWS_PER_SUB + i * EB, EB), :])
        return carry

    lax.fori_loop(0, ROWS_PER_SUB // EB, zcopy, 0)


@functools.partial(
    pl.kernel,
    out_type=jax.ShapeDtypeStruct((NC, N_PAD, DEG_W), jnp.float32),
    mesh=_MESH,
    scratch_types=[
        pltpu.VMEM((EB,), jnp.float32),
        pltpu.VMEM((EB,), jnp.int32),
        pltpu.VMEM((EB, DEG_W), jnp.float32),
        pltpu.VMEM_SHARED((N_PAD, DEG_W), jnp.float32),
    ],
    compiler_params=_SC_PARAMS,
)
def _deg_kernel(ew_hbm, dst_hbm, out_hbm, ewv, dstv, rows, acc):
    c = lax.axis_index("c")
    s = lax.axis_index("s")
    wid = s * NC + c
    _zero_acc(rows, acc, s, DEG_W)
    plsc.subcore_barrier()

    def chunk(i, carry):
        off = wid * EPW + i * EB
        pltpu.sync_copy(ew_hbm.at[pl.ds(off, EB)], ewv)
        pltpu.sync_copy(dst_hbm.at[pl.ds(off, EB)], dstv)

        def fill(blk, c2):
            wv = ewv[pl.ds(blk * 16, 16)]
            for k in range(16):
                rows[blk * 16 + k, :] = jnp.full((DEG_W,), wv[k], jnp.float32)
            return c2

        lax.fori_loop(0, EB // 16, fill, 0)
        pltpu.sync_copy(rows, acc.at[dstv], add=True)
        return carry

    lax.fori_loop(0, NCHUNK, chunk, 0)
    plsc.subcore_barrier()
    pltpu.sync_copy(
        acc.at[pl.ds(s * ROWS_PER_SUB, ROWS_PER_SUB), :],
        out_hbm.at[c, pl.ds(s * ROWS_PER_SUB, ROWS_PER_SUB), :],
    )


def _make_msg_kernel(width):
    @functools.partial(
        pl.kernel,
        out_type=jax.ShapeDtypeStruct((NC, N_PAD, width), jnp.float32),
        mesh=_MESH,
        scratch_types=[
            pltpu.VMEM((EB,), jnp.int32),
            pltpu.VMEM((EB,), jnp.int32),
            pltpu.VMEM((EB,), jnp.float32),
            pltpu.VMEM((EB, width), jnp.float32),
            pltpu.VMEM_SHARED((N_PAD, width), jnp.float32),
            pltpu.SemaphoreType.DMA,
        ],
        compiler_params=_SC_PARAMS,
    )
    def msg(g_hbm, src_hbm, dst_hbm, ew_hbm, out_hbm,
            srcv, dstv, ewv, rows, acc, sem):
        c = lax.axis_index("c")
        s = lax.axis_index("s")
        wid = s * NC + c
        _zero_acc(rows, acc, s, width)
        plsc.subcore_barrier()

        def chunk(i, carry):
            off = wid * EPW + i * EB
            pltpu.sync_copy(src_hbm.at[pl.ds(off, EB)], srcv)
            pltpu.sync_copy(dst_hbm.at[pl.ds(off, EB)], dstv)
            pltpu.sync_copy(ew_hbm.at[pl.ds(off, EB)], ewv)
            pltpu.async_copy(g_hbm.at[srcv], rows, sem).wait()

            def scale(blk, c2):
                wv = ewv[pl.ds(blk * 16, 16)]
                for k in range(16):
                    e = blk * 16 + k
                    w = wv[k]
                    for j in range(width // 16):
                        rows[e, pl.ds(j * 16, 16)] = (
                            rows[e, pl.ds(j * 16, 16)] * w)
                return c2

            lax.fori_loop(0, EB // 16, scale, 0)
            pltpu.sync_copy(rows, acc.at[dstv], add=True)
            return carry

        lax.fori_loop(0, NCHUNK, chunk, 0)
        plsc.subcore_barrier()
        pltpu.sync_copy(
            acc.at[pl.ds(s * ROWS_PER_SUB, ROWS_PER_SUB), :],
            out_hbm.at[c, pl.ds(s * ROWS_PER_SUB, ROWS_PER_SUB), :],
        )

    return msg


_msg128 = _make_msg_kernel(HIDDEN)
_msg48 = _make_msg_kernel(C_PAD)


_BN = 2000


def _mm1_body(x_ref, w_ref, o_ref):
    o_ref[...] = jnp.dot(x_ref[...], w_ref[...],
                         preferred_element_type=jnp.float32)


def _scale1_body(d0_ref, d1_ref, h_ref, dis_ref, g_ref):
    dis = lax.rsqrt(d0_ref[...] + d1_ref[...] + 1.0)
    dis_ref[...] = dis
    g_ref[...] = dis * h_ref[...]


def _mid_body(a0_ref, a1_ref, g1_ref, dis_ref, b1_ref, w2_ref, g2_ref):
    dis = dis_ref[...]
    h = dis * (a0_ref[...] + a1_ref[...] + g1_ref[...]) + b1_ref[...]
    h = jnp.maximum(h, 0.0)
    h2 = jnp.dot(h, w2_ref[...], preferred_element_type=jnp.float32)
    g2_ref[...] = dis * h2


def _final_body(a0_ref, a1_ref, g2_ref, dis_ref, b2_ref, o_ref):
    o48 = dis_ref[...] * (a0_ref[...] + a1_ref[...] + g2_ref[...]) + b2_ref[...]
    o = o48[:, :N_CLASSES]
    m = jnp.max(o, axis=1, keepdims=True)
    lse = m + jnp.log(jnp.sum(jnp.exp(o - m), axis=1, keepdims=True))
    o_ref[...] = o - lse


def _rows_spec(width):
    return pl.BlockSpec((_BN, width), lambda i: (i, 0))


def _full_spec(shape):
    return pl.BlockSpec(shape, lambda i: tuple(0 for _ in shape))


def kernel(x, edge_index, edge_weight, W1, b1, W2, b2):
    src = edge_index[0].astype(jnp.int32)
    dst = edge_index[1].astype(jnp.int32)
    ew = edge_weight.astype(jnp.float32)

    deg_parts = _deg_kernel(ew, dst)
    d0 = deg_parts[0, :N_NODES, 0:1]
    d1 = deg_parts[1, :N_NODES, 0:1]

    grid = N_NODES // _BN
    h1 = pl.pallas_call(
        _mm1_body,
        grid=(grid,),
        in_specs=[_rows_spec(D_FEAT), _full_spec((D_FEAT, HIDDEN))],
        out_specs=_rows_spec(HIDDEN),
        out_shape=jax.ShapeDtypeStruct((N_NODES, HIDDEN), jnp.float32),
    )(x, W1)

    dis, g1 = pl.pallas_call(
        _scale1_body,
        grid=(grid,),
        in_specs=[_rows_spec(1), _rows_spec(1), _rows_spec(HIDDEN)],
        out_specs=[_rows_spec(1), _rows_spec(HIDDEN)],
        out_shape=[
            jax.ShapeDtypeStruct((N_NODES, 1), jnp.float32),
            jax.ShapeDtypeStruct((N_NODES, HIDDEN), jnp.float32),
        ],
    )(d0, d1, h1)

    a1 = _msg128(g1, src, dst, ew)

    W2p = jnp.pad(W2, ((0, 0), (0, C_PAD - N_CLASSES)))
    b1r = b1.reshape(1, HIDDEN)
    b2r = jnp.pad(b2, (0, C_PAD - N_CLASSES)).reshape(1, C_PAD)

    g2 = pl.pallas_call(
        _mid_body,
        grid=(grid,),
        in_specs=[
            _rows_spec(HIDDEN), _rows_spec(HIDDEN), _rows_spec(HIDDEN),
            _rows_spec(1), _full_spec((1, HIDDEN)),
            _full_spec((HIDDEN, C_PAD)),
        ],
        out_specs=_rows_spec(C_PAD),
        out_shape=jax.ShapeDtypeStruct((N_NODES, C_PAD), jnp.float32),
    )(a1[0, :N_NODES], a1[1, :N_NODES], g1, dis, b1r, W2p)

    a2 = _msg48(g2, src, dst, ew)

    out = pl.pallas_call(
        _final_body,
        grid=(grid,),
        in_specs=[
            _rows_spec(C_PAD), _rows_spec(C_PAD), _rows_spec(C_PAD),
            _rows_spec(1), _full_spec((1, C_PAD)),
        ],
        out_specs=_rows_spec(N_CLASSES),
        out_shape=jax.ShapeDtypeStruct((N_NODES, N_CLASSES), jnp.float32),
    )(a2[0, :N_NODES], a2[1, :N_NODES], g2, dis, b2r)

    return out


# speedup vs baseline: 31.5374x; 2.9842x over previous
import functools

import jax
import jax.numpy as jnp
from jax import lax
from jax.experimental import pallas as pl
from jax.experimental.pallas import tpu as pltpu
from jax.experimental.pallas import tpu_sc as plsc

N_NODES = 10000
N_EDGES = 320000
D_FEAT = 128
HIDDEN = 128
N_CLASSES = 40
C_PAD = 48

NC = 2
NS = 16
NW = NC * NS
N_PAD = 10240
ROWS_PER_SUB = N_PAD // NS
EPW = N_EDGES // NW
EB = 40
NCHUNK = EPW // EB
NBUF = 5
PREF = 3
EWS = 48
DEG_W = 16

_MESH = plsc.VectorSubcoreMesh(core_axis_name="c", subcore_axis_name="s")
_SC_PARAMS = pltpu.CompilerParams(use_tc_tiling_on_sc=False)


def _zero_acc(zbuf, acc, s, width):
    def zrow(e, carry):
        for j in range(width // 16):
            zbuf[e, pl.ds(j * 16, 16)] = jnp.zeros((16,), jnp.float32)
        return carry

    lax.fori_loop(0, EB, zrow, 0)

    def zcopy(i, carry):
        pltpu.sync_copy(zbuf, acc.at[pl.ds(s * ROWS_PER_SUB + i * EB, EB), :])
        return carry

    lax.fori_loop(0, ROWS_PER_SUB // EB, zcopy, 0)


def _scale_rows(rows, ewb, b, width):
    def blk_body(blk, carry):
        wv = ewb[b, pl.ds(blk * 16, 16)]
        for k in range(16):
            e = blk * 16 + k
            w = wv[k]
            for j in range(width // 16):
                rows[b, e, pl.ds(j * 16, 16)] = (
                    rows[b, e, pl.ds(j * 16, 16)] * w)
        return carry

    lax.fori_loop(0, EB // 16, blk_body, 0)
    wv = ewb[b, pl.ds((EB // 16) * 16, 16)]
    for k in range(EB - (EB // 16) * 16):
        e = (EB // 16) * 16 + k
        w = wv[k]
        for j in range(width // 16):
            rows[b, e, pl.ds(j * 16, 16)] = rows[b, e, pl.ds(j * 16, 16)] * w


def _fill_rows(rows, ewb, b):
    def blk_body(blk, carry):
        wv = ewb[b, pl.ds(blk * 16, 16)]
        for k in range(16):
            rows[b, blk * 16 + k, :] = jnp.full((DEG_W,), wv[k], jnp.float32)
        return carry

    lax.fori_loop(0, EB // 16, blk_body, 0)
    wv = ewb[b, pl.ds((EB // 16) * 16, 16)]
    for k in range(EB - (EB // 16) * 16):
        rows[b, (EB // 16) * 16 + k, :] = jnp.full(
            (DEG_W,), wv[k], jnp.float32)


def _export(acc, out_hbm, c, s):
    pltpu.sync_copy(
        acc.at[pl.ds(s * ROWS_PER_SUB, ROWS_PER_SUB), :],
        out_hbm.at[c, pl.ds(s * ROWS_PER_SUB, ROWS_PER_SUB), :],
    )


def _make_msg_kernel(width):
    @functools.partial(
        pl.kernel,
        out_type=jax.ShapeDtypeStruct((NC, N_PAD, width), jnp.float32),
        mesh=_MESH,
        scratch_types=[
            pltpu.VMEM((NCHUNK, EB), jnp.int32),
            pltpu.VMEM((NCHUNK, EB), jnp.int32),
            pltpu.VMEM((NBUF, EWS), jnp.float32),
            pltpu.VMEM((NBUF, EB, width), jnp.float32),
            pltpu.VMEM_SHARED((N_PAD, width), jnp.float32),
            pltpu.SemaphoreType.DMA((NBUF,)),
            pltpu.SemaphoreType.DMA((NBUF,)),
        ],
        compiler_params=_SC_PARAMS,
    )
    def msg(g_hbm, src_hbm, dst_hbm, ew_hbm, out_hbm,
            src2, dst2, ewb, rows, acc, glsem, ssem):
        c = lax.axis_index("c")
        s = lax.axis_index("s")
        wid = s * NC + c
        ebase = wid * EPW

        _zero_acc(rows.at[0], acc, s, width)
        plsc.subcore_barrier()

        pltpu.sync_copy(src_hbm.at[wid], src2)
        pltpu.sync_copy(dst_hbm.at[wid], dst2)

        def fire(j, bj):
            pltpu.async_copy(ew_hbm.at[pl.ds(ebase + j * EB, EB)],
                             ewb.at[bj, pl.ds(0, EB)], glsem.at[bj])
            pltpu.async_copy(g_hbm.at[src2.at[j]], rows.at[bj],
                             glsem.at[bj])

        def wait_gl(b):
            pltpu.make_async_copy(ew_hbm.at[pl.ds(0, EB)],
                                  ewb.at[b, pl.ds(0, EB)],
                                  glsem.at[b]).wait()
            pltpu.make_async_copy(g_hbm.at[pl.ds(0, EB), :], rows.at[b],
                                  glsem.at[b]).wait()

        def wait_s(b):
            pltpu.make_async_copy(g_hbm.at[pl.ds(0, EB), :], rows.at[b],
                                  ssem.at[b]).wait()

        for j in range(PREF):
            fire(j, j)

        def outer(u, carry):
            for b in range(NBUF):
                i = u * NBUF + b
                j = i + PREF
                bj = (b + PREF) % NBUF

                @pl.when(j < NCHUNK)
                def _():
                    @pl.when(j >= NBUF)
                    def _():
                        wait_s(bj)

                    fire(j, bj)

                wait_gl(b)
                _scale_rows(rows, ewb, b, width)
                pltpu.async_copy(rows.at[b], acc.at[dst2.at[i]],
                                 ssem.at[b], add=True)
            return carry

        lax.fori_loop(0, NCHUNK // NBUF, outer, 0)
        for b in range(NBUF):
            wait_s(b)

        plsc.subcore_barrier()
        _export(acc, out_hbm, c, s)

    return msg


@functools.partial(
    pl.kernel,
    out_type=jax.ShapeDtypeStruct((NC, N_PAD, DEG_W), jnp.float32),
    mesh=_MESH,
    scratch_types=[
        pltpu.VMEM((NCHUNK, EB), jnp.int32),
        pltpu.VMEM((NBUF, EWS), jnp.float32),
        pltpu.VMEM((NBUF, EB, DEG_W), jnp.float32),
        pltpu.VMEM_SHARED((N_PAD, DEG_W), jnp.float32),
        pltpu.SemaphoreType.DMA((NBUF,)),
        pltpu.SemaphoreType.DMA((NBUF,)),
    ],
    compiler_params=_SC_PARAMS,
)
def _deg_kernel(ew_hbm, dst_hbm, out_hbm, dst2, ewb, rows, acc, lsem, ssem):
    c = lax.axis_index("c")
    s = lax.axis_index("s")
    wid = s * NC + c
    ebase = wid * EPW

    _zero_acc(rows.at[0], acc, s, DEG_W)
    plsc.subcore_barrier()

    pltpu.sync_copy(dst_hbm.at[wid], dst2)

    def fire(j, bj):
        pltpu.async_copy(ew_hbm.at[pl.ds(ebase + j * EB, EB)],
                         ewb.at[bj, pl.ds(0, EB)], lsem.at[bj])

    def wait_l(b):
        pltpu.make_async_copy(ew_hbm.at[pl.ds(0, EB)],
                              ewb.at[b, pl.ds(0, EB)], lsem.at[b]).wait()

    def wait_s(b):
        pltpu.make_async_copy(out_hbm.at[0, pl.ds(0, EB), :],
                              rows.at[b], ssem.at[b]).wait()

    for j in range(PREF):
        fire(j, j)

    def outer(u, carry):
        for b in range(NBUF):
            i = u * NBUF + b
            j = i + PREF
            bj = (b + PREF) % NBUF

            @pl.when(j < NCHUNK)
            def _():
                fire(j, bj)

            wait_l(b)

            @pl.when(i >= NBUF)
            def _():
                wait_s(b)

            _fill_rows(rows, ewb, b)
            pltpu.async_copy(rows.at[b], acc.at[dst2.at[i]],
                             ssem.at[b], add=True)
        return carry

    lax.fori_loop(0, NCHUNK // NBUF, outer, 0)
    for b in range(NBUF):
        wait_s(b)

    plsc.subcore_barrier()
    _export(acc, out_hbm, c, s)


_msg128 = _make_msg_kernel(HIDDEN)
_msg48 = _make_msg_kernel(C_PAD)


_BN = 2000


def _mm1_body(x_ref, w_ref, o_ref):
    o_ref[...] = jnp.dot(x_ref[...], w_ref[...],
                         preferred_element_type=jnp.float32)


def _scale1_body(d0_ref, d1_ref, h_ref, dis_ref, g_ref):
    dis = lax.rsqrt(d0_ref[...] + d1_ref[...] + 1.0)
    dis_ref[...] = dis
    g_ref[...] = dis * h_ref[...]


def _mid_body(a0_ref, a1_ref, g1_ref, dis_ref, b1_ref, w2_ref, g2_ref):
    dis = dis_ref[...]
    h = dis * (a0_ref[...] + a1_ref[...] + g1_ref[...]) + b1_ref[...]
    h = jnp.maximum(h, 0.0)
    h2 = jnp.dot(h, w2_ref[...], preferred_element_type=jnp.float32)
    g2_ref[...] = dis * h2


def _final_body(a0_ref, a1_ref, g2_ref, dis_ref, b2_ref, o_ref):
    o48 = dis_ref[...] * (a0_ref[...] + a1_ref[...] + g2_ref[...]) + b2_ref[...]
    o = o48[:, :N_CLASSES]
    m = jnp.max(o, axis=1, keepdims=True)
    lse = m + jnp.log(jnp.sum(jnp.exp(o - m), axis=1, keepdims=True))
    o_ref[...] = o - lse


def _rows_spec(width):
    return pl.BlockSpec((_BN, width), lambda i: (i, 0))


def _full_spec(shape):
    return pl.BlockSpec(shape, lambda i: tuple(0 for _ in shape))


def kernel(x, edge_index, edge_weight, W1, b1, W2, b2):
    src = edge_index[0].astype(jnp.int32).reshape(NW, NCHUNK, EB)
    dst = edge_index[1].astype(jnp.int32).reshape(NW, NCHUNK, EB)
    ew = edge_weight.astype(jnp.float32)

    deg_parts = _deg_kernel(ew, dst)
    d0 = deg_parts[0, :N_NODES, 0:1]
    d1 = deg_parts[1, :N_NODES, 0:1]

    grid = N_NODES // _BN
    h1 = pl.pallas_call(
        _mm1_body,
        grid=(grid,),
        in_specs=[_rows_spec(D_FEAT), _full_spec((D_FEAT, HIDDEN))],
        out_specs=_rows_spec(HIDDEN),
        out_shape=jax.ShapeDtypeStruct((N_NODES, HIDDEN), jnp.float32),
    )(x, W1)

    dis, g1 = pl.pallas_call(
        _scale1_body,
        grid=(grid,),
        in_specs=[_rows_spec(1), _rows_spec(1), _rows_spec(HIDDEN)],
        out_specs=[_rows_spec(1), _rows_spec(HIDDEN)],
        out_shape=[
            jax.ShapeDtypeStruct((N_NODES, 1), jnp.float32),
            jax.ShapeDtypeStruct((N_NODES, HIDDEN), jnp.float32),
        ],
    )(d0, d1, h1)

    a1 = _msg128(g1, src, dst, ew)

    W2p = jnp.pad(W2, ((0, 0), (0, C_PAD - N_CLASSES)))
    b1r = b1.reshape(1, HIDDEN)
    b2r = jnp.pad(b2, (0, C_PAD - N_CLASSES)).reshape(1, C_PAD)

    g2 = pl.pallas_call(
        _mid_body,
        grid=(grid,),
        in_specs=[
            _rows_spec(HIDDEN), _rows_spec(HIDDEN), _rows_spec(HIDDEN),
            _rows_spec(1), _full_spec((1, HIDDEN)),
            _full_spec((HIDDEN, C_PAD)),
        ],
        out_specs=_rows_spec(C_PAD),
        out_shape=jax.ShapeDtypeStruct((N_NODES, C_PAD), jnp.float32),
    )(a1[0, :N_NODES], a1[1, :N_NODES], g1, dis, b1r, W2p)

    a2 = _msg48(g2, src, dst, ew)

    out = pl.pallas_call(
        _final_body,
        grid=(grid,),
        in_specs=[
            _rows_spec(C_PAD), _rows_spec(C_PAD), _rows_spec(C_PAD),
            _rows_spec(1), _full_spec((1, C_PAD)),
        ],
        out_specs=_rows_spec(N_CLASSES),
        out_shape=jax.ShapeDtypeStruct((N_NODES, N_CLASSES), jnp.float32),
    )(a2[0, :N_NODES], a2[1, :N_NODES], g2, dis, b2r)

    return out


# speedup vs baseline: 31.5736x; 1.0011x over previous
import functools

import jax
import jax.numpy as jnp
from jax import lax
from jax.experimental import pallas as pl
from jax.experimental.pallas import tpu as pltpu
from jax.experimental.pallas import tpu_sc as plsc

N_NODES = 10000
N_EDGES = 320000
D_FEAT = 128
HIDDEN = 128
N_CLASSES = 40
C_PAD = 48

NC = 2
NS = 16
NW = NC * NS
N_PAD = 10240
ROWS_PER_SUB = N_PAD // NS
EPW = N_EDGES // NW
EB = 40
NCHUNK = EPW // EB
NBUF = 5
PREF = 3
EWS = 48
DEG_W = 16

_MESH = plsc.VectorSubcoreMesh(core_axis_name="c", subcore_axis_name="s")
_SC_PARAMS = pltpu.CompilerParams(use_tc_tiling_on_sc=False)


def _zero_acc(zbuf, acc, s, width):
    def zrow(e, carry):
        for j in range(width // 16):
            zbuf[e, pl.ds(j * 16, 16)] = jnp.zeros((16,), jnp.float32)
        return carry

    lax.fori_loop(0, EB, zrow, 0)

    def zcopy(i, carry):
        pltpu.sync_copy(zbuf, acc.at[pl.ds(s * ROWS_PER_SUB + i * EB, EB), :])
        return carry

    lax.fori_loop(0, ROWS_PER_SUB // EB, zcopy, 0)


def _scale_rows(rows, ewb, b, width):
    def blk_body(blk, carry):
        wv = ewb[b, pl.ds(blk * 16, 16)]
        for k in range(16):
            e = blk * 16 + k
            w = wv[k]
            for j in range(width // 16):
                rows[b, e, pl.ds(j * 16, 16)] = (
                    rows[b, e, pl.ds(j * 16, 16)] * w)
        return carry

    lax.fori_loop(0, EB // 16, blk_body, 0)
    wv = ewb[b, pl.ds((EB // 16) * 16, 16)]
    for k in range(EB - (EB // 16) * 16):
        e = (EB // 16) * 16 + k
        w = wv[k]
        for j in range(width // 16):
            rows[b, e, pl.ds(j * 16, 16)] = rows[b, e, pl.ds(j * 16, 16)] * w


def _fill_rows(rows, ewb, b):
    def blk_body(blk, carry):
        wv = ewb[b, pl.ds(blk * 16, 16)]
        for k in range(16):
            rows[b, blk * 16 + k, :] = jnp.full((DEG_W,), wv[k], jnp.float32)
        return carry

    lax.fori_loop(0, EB // 16, blk_body, 0)
    wv = ewb[b, pl.ds((EB // 16) * 16, 16)]
    for k in range(EB - (EB // 16) * 16):
        rows[b, (EB // 16) * 16 + k, :] = jnp.full(
            (DEG_W,), wv[k], jnp.float32)


def _export(acc, out_hbm, c, s):
    pltpu.sync_copy(
        acc.at[pl.ds(s * ROWS_PER_SUB, ROWS_PER_SUB), :],
        out_hbm.at[c, pl.ds(s * ROWS_PER_SUB, ROWS_PER_SUB), :],
    )


def _make_msg_kernel(width, table_in_spmem=False):
    scratch = [
        pltpu.VMEM((NCHUNK, EB), jnp.int32),
        pltpu.VMEM((NCHUNK, EB), jnp.int32),
        pltpu.VMEM((NBUF, EWS), jnp.float32),
        pltpu.VMEM((NBUF, EB, width), jnp.float32),
        pltpu.VMEM_SHARED((N_PAD, width), jnp.float32),
        pltpu.SemaphoreType.DMA((NBUF,)),
        pltpu.SemaphoreType.DMA((NBUF,)),
    ]
    if table_in_spmem:
        scratch.append(pltpu.VMEM_SHARED((N_PAD, width), jnp.float32))

    @functools.partial(
        pl.kernel,
        out_type=jax.ShapeDtypeStruct((NC, N_PAD, width), jnp.float32),
        mesh=_MESH,
        scratch_types=scratch,
        compiler_params=_SC_PARAMS,
    )
    def msg(g_hbm, src_hbm, dst_hbm, ew_hbm, out_hbm,
            src2, dst2, ewb, rows, acc, glsem, ssem, *maybe_table):
        c = lax.axis_index("c")
        s = lax.axis_index("s")
        wid = s * NC + c
        ebase = wid * EPW

        _zero_acc(rows.at[0], acc, s, width)
        if table_in_spmem:
            table = maybe_table[0]

            def stage(i, carry):
                base = s * (N_NODES // NS) + i * EB
                pltpu.sync_copy(g_hbm.at[pl.ds(base, EB), :], rows.at[1])
                pltpu.sync_copy(rows.at[1], table.at[pl.ds(base, EB), :])
                return carry

            lax.fori_loop(0, (N_NODES // NS) // EB, stage, 0)
            rem = (N_NODES // NS) % EB
            if rem:
                base = s * (N_NODES // NS) + ((N_NODES // NS) // EB) * EB
                pltpu.sync_copy(g_hbm.at[pl.ds(base, rem), :],
                                rows.at[1, pl.ds(0, rem), :])
                pltpu.sync_copy(rows.at[1, pl.ds(0, rem), :],
                                table.at[pl.ds(base, rem), :])
            g_src = table
        else:
            g_src = g_hbm
        plsc.subcore_barrier()

        pltpu.sync_copy(src_hbm.at[wid], src2)
        pltpu.sync_copy(dst_hbm.at[wid], dst2)

        def fire(j, bj):
            pltpu.async_copy(ew_hbm.at[pl.ds(ebase + j * EB, EB)],
                             ewb.at[bj, pl.ds(0, EB)], glsem.at[bj])
            pltpu.async_copy(g_src.at[src2.at[j]], rows.at[bj],
                             glsem.at[bj])

        def wait_gl(b):
            pltpu.make_async_copy(ew_hbm.at[pl.ds(0, EB)],
                                  ewb.at[b, pl.ds(0, EB)],
                                  glsem.at[b]).wait()
            pltpu.make_async_copy(g_hbm.at[pl.ds(0, EB), :], rows.at[b],
                                  glsem.at[b]).wait()

        def wait_s(b):
            pltpu.make_async_copy(g_hbm.at[pl.ds(0, EB), :], rows.at[b],
                                  ssem.at[b]).wait()

        for j in range(PREF):
            fire(j, j)

        def outer(u, carry):
            for b in range(NBUF):
                i = u * NBUF + b
                j = i + PREF
                bj = (b + PREF) % NBUF

                @pl.when(j < NCHUNK)
                def _():
                    @pl.when(j >= NBUF)
                    def _():
                        wait_s(bj)

                    fire(j, bj)

                wait_gl(b)
                _scale_rows(rows, ewb, b, width)
                pltpu.async_copy(rows.at[b], acc.at[dst2.at[i]],
                                 ssem.at[b], add=True)
            return carry

        lax.fori_loop(0, NCHUNK // NBUF, outer, 0)
        for b in range(NBUF):
            wait_s(b)

        plsc.subcore_barrier()
        _export(acc, out_hbm, c, s)

    return msg


@functools.partial(
    pl.kernel,
    out_type=jax.ShapeDtypeStruct((NC, N_PAD, DEG_W), jnp.float32),
    mesh=_MESH,
    scratch_types=[
        pltpu.VMEM((NCHUNK, EB), jnp.int32),
        pltpu.VMEM((NBUF, EWS), jnp.float32),
        pltpu.VMEM((NBUF, EB, DEG_W), jnp.float32),
        pltpu.VMEM_SHARED((N_PAD, DEG_W), jnp.float32),
        pltpu.SemaphoreType.DMA((NBUF,)),
        pltpu.SemaphoreType.DMA((NBUF,)),
    ],
    compiler_params=_SC_PARAMS,
)
def _deg_kernel(ew_hbm, dst_hbm, out_hbm, dst2, ewb, rows, acc, lsem, ssem):
    c = lax.axis_index("c")
    s = lax.axis_index("s")
    wid = s * NC + c
    ebase = wid * EPW

    _zero_acc(rows.at[0], acc, s, DEG_W)
    plsc.subcore_barrier()

    pltpu.sync_copy(dst_hbm.at[wid], dst2)

    def fire(j, bj):
        pltpu.async_copy(ew_hbm.at[pl.ds(ebase + j * EB, EB)],
                         ewb.at[bj, pl.ds(0, EB)], lsem.at[bj])

    def wait_l(b):
        pltpu.make_async_copy(ew_hbm.at[pl.ds(0, EB)],
                              ewb.at[b, pl.ds(0, EB)], lsem.at[b]).wait()

    def wait_s(b):
        pltpu.make_async_copy(out_hbm.at[0, pl.ds(0, EB), :],
                              rows.at[b], ssem.at[b]).wait()

    for j in range(PREF):
        fire(j, j)

    def outer(u, carry):
        for b in range(NBUF):
            i = u * NBUF + b
            j = i + PREF
            bj = (b + PREF) % NBUF

            @pl.when(j < NCHUNK)
            def _():
                fire(j, bj)

            wait_l(b)

            @pl.when(i >= NBUF)
            def _():
                wait_s(b)

            _fill_rows(rows, ewb, b)
            pltpu.async_copy(rows.at[b], acc.at[dst2.at[i]],
                             ssem.at[b], add=True)
        return carry

    lax.fori_loop(0, NCHUNK // NBUF, outer, 0)
    for b in range(NBUF):
        wait_s(b)

    plsc.subcore_barrier()
    _export(acc, out_hbm, c, s)


_msg128 = _make_msg_kernel(HIDDEN)
_msg48 = _make_msg_kernel(C_PAD)


_BN = 2000


def _mm1_body(x_ref, w_ref, d0_ref, d1_ref, dis_ref, g_ref):
    h = jnp.dot(x_ref[...], w_ref[...], preferred_element_type=jnp.float32)
    dis = lax.rsqrt(d0_ref[...] + d1_ref[...] + 1.0)
    dis_ref[...] = dis
    g_ref[...] = dis * h


def _mid_body(a0_ref, a1_ref, g1_ref, dis_ref, b1_ref, w2_ref, g2_ref):
    dis = dis_ref[...]
    h = dis * (a0_ref[...] + a1_ref[...] + g1_ref[...]) + b1_ref[...]
    h = jnp.maximum(h, 0.0)
    h2 = jnp.dot(h, w2_ref[...], preferred_element_type=jnp.float32)
    g2_ref[...] = dis * h2


def _final_body(a0_ref, a1_ref, g2_ref, dis_ref, b2_ref, o_ref):
    o48 = dis_ref[...] * (a0_ref[...] + a1_ref[...] + g2_ref[...]) + b2_ref[...]
    o = o48[:, :N_CLASSES]
    m = jnp.max(o, axis=1, keepdims=True)
    lse = m + jnp.log(jnp.sum(jnp.exp(o - m), axis=1, keepdims=True))
    o_ref[...] = o - lse


def _rows_spec(width):
    return pl.BlockSpec((_BN, width), lambda i: (i, 0))


def _full_spec(shape):
    return pl.BlockSpec(shape, lambda i: tuple(0 for _ in shape))


def kernel(x, edge_index, edge_weight, W1, b1, W2, b2):
    src = edge_index[0].astype(jnp.int32).reshape(NW, NCHUNK, EB)
    dst = edge_index[1].astype(jnp.int32).reshape(NW, NCHUNK, EB)
    ew = edge_weight.astype(jnp.float32)

    deg_parts = _deg_kernel(ew, dst)
    d0 = deg_parts[0, :N_NODES, 0:1]
    d1 = deg_parts[1, :N_NODES, 0:1]

    grid = N_NODES // _BN
    dis, g1 = pl.pallas_call(
        _mm1_body,
        grid=(grid,),
        in_specs=[
            _rows_spec(D_FEAT), _full_spec((D_FEAT, HIDDEN)),
            _rows_spec(1), _rows_spec(1),
        ],
        out_specs=[_rows_spec(1), _rows_spec(HIDDEN)],
        out_shape=[
            jax.ShapeDtypeStruct((N_NODES, 1), jnp.float32),
            jax.ShapeDtypeStruct((N_NODES, HIDDEN), jnp.float32),
        ],
    )(x, W1, d0, d1)

    a1 = _msg128(g1, src, dst, ew)

    W2p = jnp.pad(W2, ((0, 0), (0, C_PAD - N_CLASSES)))
    b1r = b1.reshape(1, HIDDEN)
    b2r = jnp.pad(b2, (0, C_PAD - N_CLASSES)).reshape(1, C_PAD)

    g2 = pl.pallas_call(
        _mid_body,
        grid=(grid,),
        in_specs=[
            _rows_spec(HIDDEN), _rows_spec(HIDDEN), _rows_spec(HIDDEN),
            _rows_spec(1), _full_spec((1, HIDDEN)),
            _full_spec((HIDDEN, C_PAD)),
        ],
        out_specs=_rows_spec(C_PAD),
        out_shape=jax.ShapeDtypeStruct((N_NODES, C_PAD), jnp.float32),
    )(a1[0, :N_NODES], a1[1, :N_NODES], g1, dis, b1r, W2p)

    a2 = _msg48(g2, src, dst, ew)

    out = pl.pallas_call(
        _final_body,
        grid=(grid,),
        in_specs=[
            _rows_spec(C_PAD), _rows_spec(C_PAD), _rows_spec(C_PAD),
            _rows_spec(1), _full_spec((1, C_PAD)),
        ],
        out_specs=_rows_spec(N_CLASSES),
        out_shape=jax.ShapeDtypeStruct((N_NODES, N_CLASSES), jnp.float32),
    )(a2[0, :N_NODES], a2[1, :N_NODES], g2, dis, b2r)

    return out


# speedup vs baseline: 34.3752x; 1.0887x over previous
import functools

import jax
import jax.numpy as jnp
from jax import lax
from jax.experimental import pallas as pl
from jax.experimental.pallas import tpu as pltpu
from jax.experimental.pallas import tpu_sc as plsc

N_NODES = 10000
N_EDGES = 320000
D_FEAT = 128
HIDDEN = 128
N_CLASSES = 40
C_PAD = 48

NC = 2
NS = 16
NW = NC * NS
N_PAD = 10240
ROWS_PER_SUB = N_PAD // NS
EPW = N_EDGES // NW
NBUF = 5
PREF = 3
EB_WIDE = 40
EB_NARROW = 80
DEG_W = 16

_MESH = plsc.VectorSubcoreMesh(core_axis_name="c", subcore_axis_name="s")
_SC_PARAMS = pltpu.CompilerParams(use_tc_tiling_on_sc=False)


def _ew_slot(eb):
    return eb if eb % 16 == 0 else (eb // 16) * 16 + 16


def _zero_acc(zbuf, acc, s, eb, width):
    def zrow(e, carry):
        for j in range(width // 16):
            zbuf[e, pl.ds(j * 16, 16)] = jnp.zeros((16,), jnp.float32)
        return carry

    lax.fori_loop(0, eb, zrow, 0)

    def zcopy(i, carry):
        pltpu.sync_copy(zbuf, acc.at[pl.ds(s * ROWS_PER_SUB + i * eb, eb), :])
        return carry

    lax.fori_loop(0, ROWS_PER_SUB // eb, zcopy, 0)


def _scale_rows(rows, ewb, b, eb, width):
    def blk_body(blk, carry):
        wv = ewb[b, pl.ds(blk * 16, 16)]
        for k in range(16):
            e = blk * 16 + k
            w = wv[k]
            for j in range(width // 16):
                rows[b, e, pl.ds(j * 16, 16)] = (
                    rows[b, e, pl.ds(j * 16, 16)] * w)
        return carry

    lax.fori_loop(0, eb // 16, blk_body, 0)
    if eb % 16:
        wv = ewb[b, pl.ds((eb // 16) * 16, 16)]
        for k in range(eb % 16):
            e = (eb // 16) * 16 + k
            w = wv[k]
            for j in range(width // 16):
                rows[b, e, pl.ds(j * 16, 16)] = (
                    rows[b, e, pl.ds(j * 16, 16)] * w)


def _fill_rows(rows, ewb, b, eb):
    def blk_body(blk, carry):
        wv = ewb[b, pl.ds(blk * 16, 16)]
        for k in range(16):
            rows[b, blk * 16 + k, :] = jnp.full((DEG_W,), wv[k], jnp.float32)
        return carry

    lax.fori_loop(0, eb // 16, blk_body, 0)
    if eb % 16:
        wv = ewb[b, pl.ds((eb // 16) * 16, 16)]
        for k in range(eb % 16):
            rows[b, (eb // 16) * 16 + k, :] = jnp.full(
                (DEG_W,), wv[k], jnp.float32)


def _export(acc, out_hbm, c, s):
    pltpu.sync_copy(
        acc.at[pl.ds(s * ROWS_PER_SUB, ROWS_PER_SUB), :],
        out_hbm.at[c, pl.ds(s * ROWS_PER_SUB, ROWS_PER_SUB), :],
    )


def _make_msg_kernel(width, eb):
    nchunk = EPW // eb
    ews = _ew_slot(eb)

    @functools.partial(
        pl.kernel,
        out_type=jax.ShapeDtypeStruct((NC, N_PAD, width), jnp.float32),
        mesh=_MESH,
        scratch_types=[
            pltpu.VMEM((nchunk, eb), jnp.int32),
            pltpu.VMEM((nchunk, eb), jnp.int32),
            pltpu.VMEM((NBUF, ews), jnp.float32),
            pltpu.VMEM((NBUF, eb, width), jnp.float32),
            pltpu.VMEM_SHARED((N_PAD, width), jnp.float32),
            pltpu.SemaphoreType.DMA((NBUF,)),
            pltpu.SemaphoreType.DMA((NBUF,)),
        ],
        compiler_params=_SC_PARAMS,
    )
    def msg(g_hbm, src_hbm, dst_hbm, ew_hbm, out_hbm,
            src2, dst2, ewb, rows, acc, glsem, ssem):
        c = lax.axis_index("c")
        s = lax.axis_index("s")
        wid = s * NC + c
        ebase = wid * EPW

        _zero_acc(rows.at[0], acc, s, eb, width)
        plsc.subcore_barrier()

        pltpu.sync_copy(src_hbm.at[wid], src2)
        pltpu.sync_copy(dst_hbm.at[wid], dst2)

        def fire(j, bj):
            pltpu.async_copy(ew_hbm.at[pl.ds(ebase + j * eb, eb)],
                             ewb.at[bj, pl.ds(0, eb)], glsem.at[bj])
            pltpu.async_copy(g_hbm.at[src2.at[j]], rows.at[bj],
                             glsem.at[bj])

        def wait_gl(b):
            pltpu.make_async_copy(ew_hbm.at[pl.ds(0, eb)],
                                  ewb.at[b, pl.ds(0, eb)],
                                  glsem.at[b]).wait()
            pltpu.make_async_copy(g_hbm.at[pl.ds(0, eb), :], rows.at[b],
                                  glsem.at[b]).wait()

        def wait_s(b):
            pltpu.make_async_copy(g_hbm.at[pl.ds(0, eb), :], rows.at[b],
                                  ssem.at[b]).wait()

        for j in range(PREF):
            fire(j, j)

        def outer(u, carry):
            for b in range(NBUF):
                i = u * NBUF + b
                j = i + PREF
                bj = (b + PREF) % NBUF

                @pl.when(j < nchunk)
                def _():
                    @pl.when(j >= NBUF)
                    def _():
                        wait_s(bj)

                    fire(j, bj)

                wait_gl(b)
                _scale_rows(rows, ewb, b, eb, width)
                pltpu.async_copy(rows.at[b], acc.at[dst2.at[i]],
                                 ssem.at[b], add=True)
            return carry

        lax.fori_loop(0, nchunk // NBUF, outer, 0)
        for b in range(NBUF):
            wait_s(b)

        plsc.subcore_barrier()
        _export(acc, out_hbm, c, s)

    return msg


def _make_deg_kernel(eb):
    nchunk = EPW // eb
    ews = _ew_slot(eb)

    @functools.partial(
        pl.kernel,
        out_type=jax.ShapeDtypeStruct((NC, N_PAD, DEG_W), jnp.float32),
        mesh=_MESH,
        scratch_types=[
            pltpu.VMEM((nchunk, eb), jnp.int32),
            pltpu.VMEM((NBUF, ews), jnp.float32),
            pltpu.VMEM((NBUF, eb, DEG_W), jnp.float32),
            pltpu.VMEM_SHARED((N_PAD, DEG_W), jnp.float32),
            pltpu.SemaphoreType.DMA((NBUF,)),
            pltpu.SemaphoreType.DMA((NBUF,)),
        ],
        compiler_params=_SC_PARAMS,
    )
    def deg(ew_hbm, dst_hbm, out_hbm, dst2, ewb, rows, acc, lsem, ssem):
        c = lax.axis_index("c")
        s = lax.axis_index("s")
        wid = s * NC + c
        ebase = wid * EPW

        _zero_acc(rows.at[0], acc, s, eb, DEG_W)
        plsc.subcore_barrier()

        pltpu.sync_copy(dst_hbm.at[wid], dst2)

        def fire(j, bj):
            pltpu.async_copy(ew_hbm.at[pl.ds(ebase + j * eb, eb)],
                             ewb.at[bj, pl.ds(0, eb)], lsem.at[bj])

        def wait_l(b):
            pltpu.make_async_copy(ew_hbm.at[pl.ds(0, eb)],
                                  ewb.at[b, pl.ds(0, eb)],
                                  lsem.at[b]).wait()

        def wait_s(b):
            pltpu.make_async_copy(out_hbm.at[0, pl.ds(0, eb), :],
                                  rows.at[b], ssem.at[b]).wait()

        for j in range(PREF):
            fire(j, j)

        def outer(u, carry):
            for b in range(NBUF):
                i = u * NBUF + b
                j = i + PREF
                bj = (b + PREF) % NBUF

                @pl.when(j < nchunk)
                def _():
                    fire(j, bj)

                wait_l(b)

                @pl.when(i >= NBUF)
                def _():
                    wait_s(b)

                _fill_rows(rows, ewb, b, eb)
                pltpu.async_copy(rows.at[b], acc.at[dst2.at[i]],
                                 ssem.at[b], add=True)
            return carry

        lax.fori_loop(0, nchunk // NBUF, outer, 0)
        for b in range(NBUF):
            wait_s(b)

        plsc.subcore_barrier()
        _export(acc, out_hbm, c, s)

    return deg


_deg_kernel = _make_deg_kernel(EB_NARROW)
_msg128 = _make_msg_kernel(HIDDEN, EB_WIDE)
_msg48 = _make_msg_kernel(C_PAD, EB_NARROW)


_BN = 2000


def _mm1_body(x_ref, w_ref, d0_ref, d1_ref, dis_ref, g_ref):
    h = jnp.dot(x_ref[...], w_ref[...], preferred_element_type=jnp.float32)
    dis = lax.rsqrt(d0_ref[...] + d1_ref[...] + 1.0)
    dis_ref[...] = dis
    g_ref[...] = dis * h


def _mid_body(a0_ref, a1_ref, g1_ref, dis_ref, b1_ref, w2_ref, g2_ref):
    dis = dis_ref[...]
    h = dis * (a0_ref[...] + a1_ref[...] + g1_ref[...]) + b1_ref[...]
    h = jnp.maximum(h, 0.0)
    h2 = jnp.dot(h, w2_ref[...], preferred_element_type=jnp.float32)
    g2_ref[...] = dis * h2


def _final_body(a0_ref, a1_ref, g2_ref, dis_ref, b2_ref, o_ref):
    o48 = dis_ref[...] * (a0_ref[...] + a1_ref[...] + g2_ref[...]) + b2_ref[...]
    o = o48[:, :N_CLASSES]
    m = jnp.max(o, axis=1, keepdims=True)
    lse = m + jnp.log(jnp.sum(jnp.exp(o - m), axis=1, keepdims=True))
    o_ref[...] = o - lse


def _rows_spec(width):
    return pl.BlockSpec((_BN, width), lambda i: (i, 0))


def _full_spec(shape):
    return pl.BlockSpec(shape, lambda i: tuple(0 for _ in shape))


def kernel(x, edge_index, edge_weight, W1, b1, W2, b2):
    src_w = edge_index[0].astype(jnp.int32).reshape(NW, EPW // EB_WIDE, EB_WIDE)
    dst_w = edge_index[1].astype(jnp.int32).reshape(NW, EPW // EB_WIDE, EB_WIDE)
    src_n = edge_index[0].astype(jnp.int32).reshape(
        NW, EPW // EB_NARROW, EB_NARROW)
    dst_n = edge_index[1].astype(jnp.int32).reshape(
        NW, EPW // EB_NARROW, EB_NARROW)
    ew = edge_weight.astype(jnp.float32)

    deg_parts = _deg_kernel(ew, dst_n)
    d0 = deg_parts[0, :N_NODES, 0:1]
    d1 = deg_parts[1, :N_NODES, 0:1]

    grid = N_NODES // _BN
    dis, g1 = pl.pallas_call(
        _mm1_body,
        grid=(grid,),
        in_specs=[
            _rows_spec(D_FEAT), _full_spec((D_FEAT, HIDDEN)),
            _rows_spec(1), _rows_spec(1),
        ],
        out_specs=[_rows_spec(1), _rows_spec(HIDDEN)],
        out_shape=[
            jax.ShapeDtypeStruct((N_NODES, 1), jnp.float32),
            jax.ShapeDtypeStruct((N_NODES, HIDDEN), jnp.float32),
        ],
    )(x, W1, d0, d1)

    a1 = _msg128(g1, src_w, dst_w, ew)

    W2p = jnp.pad(W2, ((0, 0), (0, C_PAD - N_CLASSES)))
    b1r = b1.reshape(1, HIDDEN)
    b2r = jnp.pad(b2, (0, C_PAD - N_CLASSES)).reshape(1, C_PAD)

    g2 = pl.pallas_call(
        _mid_body,
        grid=(grid,),
        in_specs=[
            _rows_spec(HIDDEN), _rows_spec(HIDDEN), _rows_spec(HIDDEN),
            _rows_spec(1), _full_spec((1, HIDDEN)),
            _full_spec((HIDDEN, C_PAD)),
        ],
        out_specs=_rows_spec(C_PAD),
        out_shape=jax.ShapeDtypeStruct((N_NODES, C_PAD), jnp.float32),
    )(a1[0, :N_NODES], a1[1, :N_NODES], g1, dis, b1r, W2p)

    a2 = _msg48(g2, src_n, dst_n, ew)

    out = pl.pallas_call(
        _final_body,
        grid=(grid,),
        in_specs=[
            _rows_spec(C_PAD), _rows_spec(C_PAD), _rows_spec(C_PAD),
            _rows_spec(1), _full_spec((1, C_PAD)),
        ],
        out_specs=_rows_spec(N_CLASSES),
        out_shape=jax.ShapeDtypeStruct((N_NODES, N_CLASSES), jnp.float32),
    )(a2[0, :N_NODES], a2[1, :N_NODES], g2, dis, b2r)

    return out
